# baseline (speedup 1.0000x reference)
import functools

import jax
import jax.numpy as jnp
from jax import lax
from jax.experimental import pallas as pl
from jax.experimental.pallas import tpu as pltpu
from jax.experimental.pallas import tpu_sc as plsc

B = 4
S = 2048
D = 768
NC = 2
NS = 16
NW = NC * NS
P = S // NW
C = 32
H = P // C
NCH = B * H
LANES = 16
NCOL = D // LANES


def _body(tok_hbm, emb_hbm, pos_hbm, out_hbm,
          pos_v, idx_all, buf0, buf1, psem, gsem0, gsem1, osem0, osem1):
    wid = lax.axis_index("s") * NC + lax.axis_index("c")
    p0 = wid * P

    bufs = (buf0, buf1)
    gsems = (gsem0, gsem1)
    osems = (osem0, osem1)

    pos_cp = pltpu.async_copy(pos_hbm.at[pl.ds(p0, P)], pos_v, psem)
    for b in range(B):
        pltpu.sync_copy(tok_hbm.at[pl.ds(b * S + p0, P)], idx_all.at[b])

    def start_gather(i, s):
        b, h = divmod(i, H)
        return pltpu.async_copy(
            emb_hbm.at[idx_all.at[b, pl.ds(h * C, C)]], bufs[s], gsems[s])

    gat_cp = [start_gather(0, 0), start_gather(1, 1)]
    out_cp = [None, None]
    pos_cp.wait()

    for i in range(NCH):
        s = i & 1
        b, h = divmod(i, H)
        gat_cp[s].wait()
        buf = bufs[s]

        def addrow(r, carry, buf=buf, h=h):
            pr = h * C + r
            for c in range(NCOL):
                sl = pl.ds(c * LANES, LANES)
                buf[r, sl] = buf[r, sl] + pos_v[pr, sl]
            return carry

        lax.fori_loop(0, C, addrow, 0)
        out_cp[s] = pltpu.async_copy(
            buf, out_hbm.at[b, pl.ds(p0 + h * C, C)], osems[s])
        if i + 2 < NCH:
            out_cp[s].wait()
            gat_cp[s] = start_gather(i + 2, s)

    out_cp[0].wait()
    out_cp[1].wait()


@functools.partial(jax.jit, static_argnames=())
def _run(tok_flat, emb, pos_emb):
    mesh = plsc.VectorSubcoreMesh(core_axis_name="c", subcore_axis_name="s")
    f = pl.kernel(
        _body,
        out_type=jax.ShapeDtypeStruct((B, S, D), jnp.float32),
        mesh=mesh,
        scratch_types=[
            pltpu.VMEM((P, D), jnp.float32),
            pltpu.VMEM((B, P), jnp.int32),
            pltpu.VMEM((C, D), jnp.float32),
            pltpu.VMEM((C, D), jnp.float32),
            pltpu.SemaphoreType.DMA,
            pltpu.SemaphoreType.DMA,
            pltpu.SemaphoreType.DMA,
            pltpu.SemaphoreType.DMA,
            pltpu.SemaphoreType.DMA,
        ],
    )
    return f(tok_flat, emb, pos_emb)


def kernel(tokens, emb, pos_emb):
    tok_flat = tokens.reshape(-1).astype(jnp.int32)
    return _run(tok_flat, emb, pos_emb)


# speedup vs baseline: 1.2461x; 1.2461x over previous
import functools

import jax
import jax.numpy as jnp
from jax import lax
from jax.experimental import pallas as pl
from jax.experimental.pallas import tpu as pltpu
from jax.experimental.pallas import tpu_sc as plsc

B = 4
S = 2048
D = 768
NC = 2
NS = 16
NW = NC * NS
P = S // NW
C = 32
H = P // C
NCH = B * H
LANES = 16
NCOL = D // LANES


NBUF = 3


def _body(tok_hbm, emb_hbm, pos_hbm, out_hbm,
          pos_v, idx_all, buf0, buf1, buf2,
          psem, gsem0, gsem1, gsem2, osem0, osem1, osem2):
    wid = lax.axis_index("s") * NC + lax.axis_index("c")
    p0 = wid * P

    bufs = (buf0, buf1, buf2)
    gsems = (gsem0, gsem1, gsem2)
    osems = (osem0, osem1, osem2)

    pos_cp = pltpu.async_copy(pos_hbm.at[pl.ds(p0, P)], pos_v, psem)
    for b in range(B):
        pltpu.sync_copy(tok_hbm.at[pl.ds(b * S + p0, P)], idx_all.at[b])

    def start_gather(i):
        b, h = divmod(i, H)
        s = i % NBUF
        return pltpu.async_copy(
            emb_hbm.at[idx_all.at[b, pl.ds(h * C, C)]], bufs[s], gsems[s])

    gat_cp = [start_gather(0), start_gather(1), None]
    out_cp = [None, None, None]
    pos_cp.wait()

    for i in range(NCH):
        s = i % NBUF
        b, h = divmod(i, H)
        g = i + 2
        if g < NCH:
            if g >= NBUF:
                out_cp[g % NBUF].wait()
            gat_cp[g % NBUF] = start_gather(g)
        gat_cp[s].wait()
        buf = bufs[s]

        @plsc.parallel_loop(0, C, 1, unroll=2)
        def _(r, buf=buf, h=h):
            pr = h * C + r
            for c in range(NCOL):
                sl = pl.ds(c * LANES, LANES)
                buf[r, sl] = buf[r, sl] + pos_v[pr, sl]

        out_cp[s] = pltpu.async_copy(
            buf, out_hbm.at[b, pl.ds(p0 + h * C, C)], osems[s])

    for i in range(NCH - NBUF, NCH):
        out_cp[i % NBUF].wait()


@functools.partial(jax.jit, static_argnames=())
def _run(tok_flat, emb, pos_emb):
    mesh = plsc.VectorSubcoreMesh(core_axis_name="c", subcore_axis_name="s")
    f = pl.kernel(
        _body,
        out_type=jax.ShapeDtypeStruct((B, S, D), jnp.float32),
        mesh=mesh,
        scratch_types=[
            pltpu.VMEM((P, D), jnp.float32),
            pltpu.VMEM((B, P), jnp.int32),
            pltpu.VMEM((C, D), jnp.float32),
            pltpu.VMEM((C, D), jnp.float32),
            pltpu.VMEM((C, D), jnp.float32),
            pltpu.SemaphoreType.DMA,
            pltpu.SemaphoreType.DMA,
            pltpu.SemaphoreType.DMA,
            pltpu.SemaphoreType.DMA,
            pltpu.SemaphoreType.DMA,
            pltpu.SemaphoreType.DMA,
            pltpu.SemaphoreType.DMA,
        ],
    )
    return f(tok_flat, emb, pos_emb)


def kernel(tokens, emb, pos_emb):
    tok_flat = tokens.reshape(-1).astype(jnp.int32)
    return _run(tok_flat, emb, pos_emb)
